# baseline (speedup 1.0000x reference)
import jax
import jax.numpy as jnp
from jax.experimental import pallas as pl
from jax.experimental.pallas import tpu as pltpu

_DIM = 64
_K = 1024
_BETA = 0.25
_L = 8
_EPS = 1e-10
_N = 16384
_B = 2048
_CHAINS = 8


def _body(z_ref, wcat_ref, dw_ref, b_ref,
          rep_ref, zdl_ref, recon_ref, perp_ref,
          d2_s, counts_s, sq_s):
    step = pl.program_id(0)
    nsteps = pl.num_programs(0)
    zb = z_ref[...]
    wcat = wcat_ref[...]

    @pl.when(step == 0)
    def _init():
        dwt = wcat[:, _K:]
        d2_s[...] = 0.5 * jnp.sum(dwt * dwt, axis=0, keepdims=True)
        counts_s[...] = jnp.zeros((1, _K), jnp.float32)
        sq_s[0] = 0.0

    both = jax.lax.dot_general(zb, wcat, (((1,), (0,)), ((), ())),
                               preferred_element_type=jnp.float32)

    e = jnp.exp(both[:, :_K] + b_ref[...])
    rep = e * (1.0 / jnp.sum(e, axis=1, keepdims=True))
    rep_ref[...] = rep

    score = both[:, _K:] - d2_s[...]

    _NET = [(0, 1), (2, 3), (4, 5), (6, 7),
            (0, 2), (1, 3), (4, 6), (5, 7),
            (1, 2), (5, 6), (0, 4), (3, 7),
            (1, 5), (2, 6),
            (1, 4), (3, 6),
            (2, 4), (3, 5),
            (3, 4)]

    def _thresh(work):
        p = [work[:, l * 128:(l + 1) * 128] for l in range(8)]
        for i, j in _NET:
            a, b = p[i], p[j]
            p[i] = jnp.maximum(a, b)
            p[j] = jnp.minimum(a, b)
        t = None
        for it in range(_L):
            t = jnp.max(p[0], axis=1, keepdims=True)
            if it < _L - 1:
                sel = p[0] >= t
                for k in range(7 - it):
                    p[k] = jnp.where(sel, p[k + 1], p[k])
        return t

    half = _B // _CHAINS
    cnt_new = counts_s[...]
    sq = 0.0
    dw = dw_ref[...]
    for h in range(_CHAINS):
        sl = slice(h * half, (h + 1) * half)
        sc_h = score[sl]
        mask = sc_h >= _thresh(sc_h)
        rep_sp = jnp.where(mask, rep[sl], 0.0)
        zdl = jax.lax.dot_general(rep_sp, dw, (((1,), (0,)), ((), ())),
                                  preferred_element_type=jnp.float32)
        zdl_ref[sl, :] = zdl
        diff = zdl - zb[sl]
        sq = sq + jnp.sum(diff * diff)
        cnt_new = cnt_new + jnp.sum(mask.astype(jnp.float32), axis=0,
                                    keepdims=True)
    sq_s[0] = sq_s[0] + sq
    counts_s[...] = cnt_new

    @pl.when(step == nsteps - 1)
    def _fin():
        cnt = counts_s[...]
        avg = cnt / jnp.sum(cnt)
        perp = jnp.exp(-jnp.sum(avg * jnp.log(avg + _EPS)))
        perp_ref[...] = perp.reshape(1, 1)
        recon = sq_s[0] * ((1.0 + _BETA) / (_N * _DIM))
        recon_ref[...] = recon.reshape(1, 1)


def _run(zf, wcat, dw, b):
    return pl.pallas_call(
        _body,
        grid=(_N // _B,),
        in_specs=[
            pl.BlockSpec((_B, _DIM), lambda i: (i, 0)),
            pl.BlockSpec((_DIM, 2 * _K), lambda i: (0, 0)),
            pl.BlockSpec((_K, _DIM), lambda i: (0, 0)),
            pl.BlockSpec((1, _K), lambda i: (0, 0)),
        ],
        out_specs=[
            pl.BlockSpec((_B, _K), lambda i: (i, 0)),
            pl.BlockSpec((_B, _DIM), lambda i: (i, 0)),
            pl.BlockSpec((1, 1), lambda i: (0, 0)),
            pl.BlockSpec((1, 1), lambda i: (0, 0)),
        ],
        out_shape=[
            jax.ShapeDtypeStruct((_N, _K), jnp.float32),
            jax.ShapeDtypeStruct((_N, _DIM), jnp.float32),
            jax.ShapeDtypeStruct((1, 1), jnp.float32),
            jax.ShapeDtypeStruct((1, 1), jnp.float32),
        ],
        scratch_shapes=[
            pltpu.VMEM((1, _K), jnp.float32),
            pltpu.VMEM((1, _K), jnp.float32),
            pltpu.SMEM((1,), jnp.float32),
        ],
    )(zf, wcat, dw, b)


def kernel(z_e, dict_w, lin_w, lin_b):
    n, c, h, w = z_e.shape
    zf = jnp.transpose(z_e, (0, 2, 3, 1)).reshape(-1, _DIM)
    wcat = jnp.concatenate([lin_w.T, dict_w.T], axis=1)
    rep, zdl, recon, perp = _run(zf, wcat, dict_w, lin_b.reshape(1, _K))
    z_st = jnp.transpose(zdl.reshape(n, h, w, c), (0, 3, 1, 2))
    return (recon.reshape(()), z_st, perp.reshape(()), rep)


# speedup vs baseline: 1.0156x; 1.0156x over previous
import jax
import jax.numpy as jnp
from jax.experimental import pallas as pl
from jax.experimental.pallas import tpu as pltpu

_DIM = 64
_K = 1024
_BETA = 0.25
_L = 8
_EPS = 1e-10
_N = 16384
_B = 2048
_CHAINS = 8


def _body(z_ref, dwt_ref, dw_ref, lwt_ref, b_ref,
          rep_ref, zdl_ref, recon_ref, perp_ref,
          d2_s, counts_s, sq_s):
    step = pl.program_id(0)
    nsteps = pl.num_programs(0)
    zb = z_ref[...]
    dwt = dwt_ref[...]

    @pl.when(step == 0)
    def _init():
        d2_s[...] = 0.5 * jnp.sum(dwt * dwt, axis=0, keepdims=True)
        counts_s[...] = jnp.zeros((1, _K), jnp.float32)
        sq_s[0] = 0.0

    logits = jax.lax.dot_general(zb, lwt_ref[...], (((1,), (0,)), ((), ())),
                                 preferred_element_type=jnp.float32)
    e = jnp.exp(logits + b_ref[...])
    rep = e * (1.0 / jnp.sum(e, axis=1, keepdims=True))
    rep_ref[...] = rep

    xz = jax.lax.dot_general(zb, dwt, (((1,), (0,)), ((), ())),
                             preferred_element_type=jnp.float32)
    score = xz - d2_s[...]

    _NET = [(0, 1), (2, 3), (4, 5), (6, 7),
            (0, 2), (1, 3), (4, 6), (5, 7),
            (1, 2), (5, 6), (0, 4), (3, 7),
            (1, 5), (2, 6),
            (1, 4), (3, 6),
            (2, 4), (3, 5),
            (3, 4)]

    def _thresh(work):
        p = [work[:, l * 128:(l + 1) * 128] for l in range(8)]
        for i, j in _NET:
            a, b = p[i], p[j]
            p[i] = jnp.maximum(a, b)
            p[j] = jnp.minimum(a, b)
        t = None
        for it in range(_L):
            t = jnp.max(p[0], axis=1, keepdims=True)
            if it < _L - 1:
                sel = p[0] >= t
                for k in range(7 - it):
                    p[k] = jnp.where(sel, p[k + 1], p[k])
        return t

    half = _B // _CHAINS
    cnt_new = counts_s[...]
    sq = 0.0
    dw = dw_ref[...]
    for h in range(_CHAINS):
        sl = slice(h * half, (h + 1) * half)
        sc_h = score[sl]
        mask = sc_h >= _thresh(sc_h)
        rep_sp = jnp.where(mask, rep[sl], 0.0)
        zdl = jax.lax.dot_general(rep_sp, dw, (((1,), (0,)), ((), ())),
                                  preferred_element_type=jnp.float32)
        zdl_ref[sl, :] = zdl
        diff = zdl - zb[sl]
        sq = sq + jnp.sum(diff * diff)
        cnt_new = cnt_new + jnp.sum(mask.astype(jnp.float32), axis=0,
                                    keepdims=True)
    sq_s[0] = sq_s[0] + sq
    counts_s[...] = cnt_new

    @pl.when(step == nsteps - 1)
    def _fin():
        cnt = counts_s[...]
        avg = cnt / jnp.sum(cnt)
        perp = jnp.exp(-jnp.sum(avg * jnp.log(avg + _EPS)))
        perp_ref[...] = perp.reshape(1, 1)
        recon = sq_s[0] * ((1.0 + _BETA) / (_N * _DIM))
        recon_ref[...] = recon.reshape(1, 1)


def _run(zf, dwt, dw, lwt, b):
    return pl.pallas_call(
        _body,
        grid=(_N // _B,),
        in_specs=[
            pl.BlockSpec((_B, _DIM), lambda i: (i, 0)),
            pl.BlockSpec((_DIM, _K), lambda i: (0, 0)),
            pl.BlockSpec((_K, _DIM), lambda i: (0, 0)),
            pl.BlockSpec((_DIM, _K), lambda i: (0, 0)),
            pl.BlockSpec((1, _K), lambda i: (0, 0)),
        ],
        out_specs=[
            pl.BlockSpec((_B, _K), lambda i: (i, 0)),
            pl.BlockSpec((_B, _DIM), lambda i: (i, 0)),
            pl.BlockSpec((1, 1), lambda i: (0, 0)),
            pl.BlockSpec((1, 1), lambda i: (0, 0)),
        ],
        out_shape=[
            jax.ShapeDtypeStruct((_N, _K), jnp.float32),
            jax.ShapeDtypeStruct((_N, _DIM), jnp.float32),
            jax.ShapeDtypeStruct((1, 1), jnp.float32),
            jax.ShapeDtypeStruct((1, 1), jnp.float32),
        ],
        scratch_shapes=[
            pltpu.VMEM((1, _K), jnp.float32),
            pltpu.VMEM((1, _K), jnp.float32),
            pltpu.SMEM((1,), jnp.float32),
        ],
    )(zf, dwt, dw, lwt, b)


def kernel(z_e, dict_w, lin_w, lin_b):
    n, c, h, w = z_e.shape
    zf = jnp.transpose(z_e, (0, 2, 3, 1)).reshape(-1, _DIM)
    rep, zdl, recon, perp = _run(zf, dict_w.T, dict_w, lin_w.T,
                                 lin_b.reshape(1, _K))
    z_st = jnp.transpose(zdl.reshape(n, h, w, c), (0, 3, 1, 2))
    return (recon.reshape(()), z_st, perp.reshape(()), rep)
